# baseline (speedup 1.0000x reference)
import jax
import jax.numpy as jnp
from jax import lax
from jax.experimental import pallas as pl
from jax.experimental.pallas import tpu as pltpu
from jax.experimental.pallas import tpu_sc as plsc

KN = 128
EXER_N = 50000

NSUB = 16
NCORE = 2
Q = 12544
CPAD = 4 * Q
ROWS_PER_TILE = Q // NSUB
CHUNK = 2048
NCHUNK = 13
EPAD = NSUB * NCHUNK * CHUNK
GARB = Q * 128
HISTW = (Q + 16) * 128
ZCH = 2512
E_DIR = 2048
E_UND = 4096


def _sc_hist_common(half, ek_src, ek_dst, dir_src, dir_dst, und_src, und_dst,
                    cek_out, cd_out, cu_out,
                    hist, hist_d, hist_u,
                    srcb0, srcb1, dstb0, dstb1, idxb0, idxb1, onesb, zb,
                    lsem0, lsem1, ssem0, ssem1):
    small = half == 0
    c = lax.axis_index("c")
    s = lax.axis_index("s")
    srcb = (srcb0, srcb1)
    dstb = (dstb0, dstb1)
    idxb = (idxb0, idxb1)
    lsem = (lsem0, lsem1)
    ssem = (ssem0, ssem1)

    zero16 = jnp.zeros((16,), jnp.float32)
    one16 = jnp.ones((16,), jnp.float32)

    def init_body(i, _):
        zb[pl.ds(i * 16, 16)] = zero16
        return 0
    lax.fori_loop(0, ZCH // 16, init_body, 0)

    def ones_body(i, _):
        onesb[pl.ds(i * 16, 16)] = one16
        return 0
    lax.fori_loop(0, CHUNK // 16, ones_body, 0)

    lane = lax.iota(jnp.int32, 16)

    def compute_idx(sb, db, ib, base):
        def row_body(r, _):
            for kk in range(8):
                o = r * 128 + kk * 16
                d16 = db[pl.ds(o, 16)]
                s16 = sb[pl.ds(o, 16)]
                if base is None:
                    idx = d16 * 128 + s16
                else:
                    loc = d16 * 128 + (s16 - EXER_N) - base
                    ok = jnp.logical_and(loc >= 0, loc < Q * 128)
                    idx = jnp.where(ok, loc, GARB + o + lane)
                ib[pl.ds(o, 16)] = idx
            return 0
        lax.fori_loop(0, 16, row_body, 0)

    def scatter_pass(src_ref, dst_ref, qbase, tile_off):
        loads = [None] * NCHUNK
        streams = [None] * NCHUNK

        def start_load(t):
            off = tile_off + t * CHUNK
            b = t % 2
            loads[t] = (
                pltpu.async_copy(src_ref.at[pl.ds(off, CHUNK)], srcb[b], lsem[b]),
                pltpu.async_copy(dst_ref.at[pl.ds(off, CHUNK)], dstb[b], lsem[b]),
            )

        start_load(0)
        for t in range(NCHUNK):
            b = t % 2
            if t + 1 < NCHUNK:
                start_load(t + 1)
            for d in loads[t]:
                d.wait()
            if t >= 2:
                streams[t - 2].wait()
            compute_idx(srcb[b], dstb[b], idxb[b], qbase)
            streams[t] = pltpu.async_copy(onesb, hist.at[idxb[b]], ssem[b],
                                          add=True)
        streams[NCHUNK - 2].wait()
        streams[NCHUNK - 1].wait()

    def scatter_small(src_ref, dst_ref, off, target):
        pltpu.sync_copy(src_ref.at[pl.ds(off, CHUNK)], srcb0)
        pltpu.sync_copy(dst_ref.at[pl.ds(off, CHUNK)], dstb0)
        compute_idx(srcb0, dstb0, idxb0, None)
        pltpu.sync_copy(onesb, target.at[idxb0], add=True)

    quarter = 2 * half + c
    qbase = quarter * (Q * 128)

    zcopies = [pltpu.async_copy(
        zb, hist.at[pl.ds(s * (HISTW // 16) + j * ZCH, ZCH)],
        lsem[j % 2]) for j in range(40)]
    for d in zcopies:
        d.wait()
    if small:
        @pl.when(jnp.logical_and(c == 0, s == 2))
        def _():
            for j in range(8):
                pltpu.sync_copy(zb.at[pl.ds(0, 2048)],
                                hist_d.at[pl.ds(j * 2048, 2048)])

        @pl.when(jnp.logical_and(c == 0, s == 3))
        def _():
            for j in range(8):
                pltpu.sync_copy(zb.at[pl.ds(0, 2048)],
                                hist_u.at[pl.ds(j * 2048, 2048)])
    plsc.subcore_barrier()

    scatter_pass(ek_src, ek_dst, qbase, s * NCHUNK * CHUNK)

    if small:
        @pl.when(jnp.logical_and(c == 0, s == 0))
        def _():
            scatter_small(dir_src, dir_dst, 0, hist_d)

        @pl.when(jnp.logical_and(c == 0, s == 1))
        def _():
            scatter_small(und_src, und_dst, 0, hist_u)
            scatter_small(und_src, und_dst, CHUNK, hist_u)
    plsc.subcore_barrier()

    nw = ROWS_PER_TILE * 128
    pltpu.sync_copy(hist.at[pl.ds(s * nw, nw)],
                    cek_out.at[pl.ds(c * Q * 128 + s * nw, nw)])
    if small:
        @pl.when(jnp.logical_and(c == 0, s == 2))
        def _():
            pltpu.sync_copy(hist_d, cd_out)

        @pl.when(jnp.logical_and(c == 0, s == 3))
        def _():
            pltpu.sync_copy(hist_u, cu_out)


def _sc_half0_body(ek_src, ek_dst, dir_src, dir_dst, und_src, und_dst,
                   cek_out, cd_out, cu_out,
                   hist, hist_d, hist_u,
                   srcb0, srcb1, dstb0, dstb1, idxb0, idxb1, onesb, zb,
                   lsem0, lsem1, ssem0, ssem1):
    _sc_hist_common(0, ek_src, ek_dst, dir_src, dir_dst, und_src, und_dst,
                    cek_out, cd_out, cu_out, hist, hist_d, hist_u,
                    srcb0, srcb1, dstb0, dstb1, idxb0, idxb1, onesb, zb,
                    lsem0, lsem1, ssem0, ssem1)


def _sc_half1_body(ek_src, ek_dst,
                   cek_out,
                   hist,
                   srcb0, srcb1, dstb0, dstb1, idxb0, idxb1, onesb, zb,
                   lsem0, lsem1, ssem0, ssem1):
    _sc_hist_common(1, ek_src, ek_dst, None, None, None, None,
                    cek_out, None, None, hist, None, None,
                    srcb0, srcb1, dstb0, dstb1, idxb0, idxb1, onesb, zb,
                    lsem0, lsem1, ssem0, ssem1)


_SC_TILE_SCRATCH = [
    pltpu.VMEM((CHUNK,), jnp.int32),
    pltpu.VMEM((CHUNK,), jnp.int32),
    pltpu.VMEM((CHUNK,), jnp.int32),
    pltpu.VMEM((CHUNK,), jnp.int32),
    pltpu.VMEM((CHUNK,), jnp.int32),
    pltpu.VMEM((CHUNK,), jnp.int32),
    pltpu.VMEM((CHUNK,), jnp.float32),
    pltpu.VMEM((ZCH,), jnp.float32),
    pltpu.SemaphoreType.DMA,
    pltpu.SemaphoreType.DMA,
    pltpu.SemaphoreType.DMA,
    pltpu.SemaphoreType.DMA,
]


def _sc_mesh():
    return plsc.VectorSubcoreMesh(core_axis_name="c", subcore_axis_name="s",
                                  num_cores=NCORE, num_subcores=NSUB)


def _sc_histograms_half0(ek_src, ek_dst, dir_src, dir_dst, und_src, und_dst):
    f32 = jnp.float32
    return pl.kernel(
        _sc_half0_body,
        out_type=(
            jax.ShapeDtypeStruct((2 * Q * 128,), f32),
            jax.ShapeDtypeStruct((KN * KN,), f32),
            jax.ShapeDtypeStruct((KN * KN,), f32),
        ),
        mesh=_sc_mesh(),
        scratch_types=[
            pltpu.VMEM_SHARED((HISTW,), f32),
            pltpu.VMEM_SHARED((KN * KN,), f32),
            pltpu.VMEM_SHARED((KN * KN,), f32),
        ] + _SC_TILE_SCRATCH,
        name="edge_histograms_sc_a",
    )(ek_src, ek_dst, dir_src, dir_dst, und_src, und_dst)


def _sc_histograms_half1(ek_src, ek_dst):
    f32 = jnp.float32
    return pl.kernel(
        _sc_half1_body,
        out_type=jax.ShapeDtypeStruct((2 * Q * 128,), f32),
        mesh=_sc_mesh(),
        scratch_types=[
            pltpu.VMEM_SHARED((HISTW,), f32),
        ] + _SC_TILE_SCRATCH,
        name="edge_histograms_sc_b",
    )(ek_src, ek_dst)


def _gat_dense(z, s_row, t_col, cnt):
    u = t_col + s_row
    f = jnp.maximum(u, jnp.float32(0.01) * u)
    a = t_col + jnp.max(s_row, axis=1, keepdims=True)
    m = jnp.maximum(a, jnp.float32(0.01) * a)
    p = cnt * jnp.exp(f - m)
    den = jnp.sum(p, axis=1, keepdims=True)
    q = jnp.dot(p, z, preferred_element_type=jnp.float32)
    return q * (jnp.float32(1.0) / (den + jnp.float32(1e-30)))


def _nt(a, b):
    return lax.dot_general(a, b, (((1,), (1,)), ((), ())),
                           preferred_element_type=jnp.float32)


def _tc_small_body(kn_ref, cd_ref, cu_ref, wdt_ref, a1d_ref, a2d_ref,
                   wut_ref, a1u_ref, a2u_ref, wekt_ref, a1ek_ref, a2ek_ref,
                   wk1a_ref, wk1b_ref, bk1_ref, wk2a_ref, wk2b_ref, bk2_ref,
                   kn_out_ref, zkek_ref, sek_ref, vek_ref):
    A = kn_ref[...]

    zd = jnp.dot(A, wdt_ref[...], preferred_element_type=jnp.float32)
    B = _gat_dense(zd, _nt(a1d_ref[...], zd),
                   jnp.dot(zd, a2d_ref[...], preferred_element_type=jnp.float32),
                   cd_ref[...])

    zu = jnp.dot(A, wut_ref[...], preferred_element_type=jnp.float32)
    C = _gat_dense(zu, _nt(a1u_ref[...], zu),
                   jnp.dot(zu, a2u_ref[...], preferred_element_type=jnp.float32),
                   cu_ref[...])

    s1 = (jnp.dot(A, wk1a_ref[...], preferred_element_type=jnp.float32)
          + jnp.dot(B, wk1b_ref[...], preferred_element_type=jnp.float32)
          + bk1_ref[...])
    s2 = (jnp.dot(A, wk2a_ref[...], preferred_element_type=jnp.float32)
          + jnp.dot(C, wk2b_ref[...], preferred_element_type=jnp.float32)
          + bk2_ref[...])
    mm = jnp.maximum(s1, s2)
    e1 = jnp.exp(s1 - mm)
    e2 = jnp.exp(s2 - mm)
    kn_out_ref[...] = A + (e1 * B + e2 * C) / (e1 + e2)

    zek = jnp.dot(A, wekt_ref[...], preferred_element_type=jnp.float32)
    zkek_ref[...] = zek
    sek_ref[...] = _nt(a1ek_ref[...], zek)
    vek_ref[...] = jnp.dot(wekt_ref[...], a2ek_ref[...],
                           preferred_element_type=jnp.float32)


def _tc_small(kn_emb, C_d, C_u, args):
    f32 = jnp.float32
    out_shape = (
        jax.ShapeDtypeStruct((KN, KN), f32),
        jax.ShapeDtypeStruct((KN, KN), f32),
        jax.ShapeDtypeStruct((1, KN), f32),
        jax.ShapeDtypeStruct((KN, 1), f32),
    )
    return pl.pallas_call(
        _tc_small_body,
        out_shape=out_shape,
        name="knowledge_layers_tc",
    )(kn_emb, C_d, C_u, *args)


BLK = 512
HBLK = 2 * Q // BLK


def _tc_big_body(x_ref, c_ref, zk_ref, s_ref, v2_ref, wb_ref, be_ref,
                 o_ref):
    X = x_ref[...]
    XV = jnp.dot(X, v2_ref[...], preferred_element_type=jnp.float32)
    t = XV[:, 0:1]
    sa = XV[:, 1:2]
    s = s_ref[...]
    u = t + s
    f = jnp.maximum(u, jnp.float32(0.01) * u)
    a = t + jnp.max(s, axis=1, keepdims=True)
    m = jnp.maximum(a, jnp.float32(0.01) * a)
    p = c_ref[...] * jnp.exp(f - m)
    den = jnp.sum(p, axis=1, keepdims=True)
    q = jnp.dot(p, zk_ref[...], preferred_element_type=jnp.float32)
    Be = q * (jnp.float32(1.0) / (den + jnp.float32(1e-30)))
    se = (sa + jnp.dot(Be, wb_ref[...], preferred_element_type=jnp.float32)
          + be_ref[...])
    o_ref[...] = X + se * Be


def _tc_big2_body(prev_ref, x_ref, c_ref, zk_ref, s_ref, v2_ref, wb_ref,
                  be_ref, o_ref):
    _tc_big_body(x_ref, c_ref, zk_ref, s_ref, v2_ref, wb_ref, be_ref, o_ref)


def _tc_big(x, C0, C1, zk_ek, s_ek, v2, web, be1):
    f32 = jnp.float32
    full = lambda shp: pl.BlockSpec(shp, lambda i: (0, 0))
    c_spec = pl.BlockSpec((BLK, KN), lambda i: (i, 0))
    lo_spec = pl.BlockSpec((BLK, KN), lambda i: (i, 0))
    hi_spec = pl.BlockSpec((BLK, KN), lambda i: (i + HBLK, 0))
    small_specs = [full((KN, KN)), full((1, KN)), full((KN, 2)),
                   full((KN, 1)), full((1, 1))]
    o1 = pl.pallas_call(
        _tc_big_body,
        grid=(HBLK,),
        in_specs=[lo_spec, c_spec] + small_specs,
        out_specs=lo_spec,
        out_shape=jax.ShapeDtypeStruct((EXER_N, KN), f32),
        name="exercise_layer_tc_a",
    )(x, C0, zk_ek, s_ek, v2, web, be1)
    return pl.pallas_call(
        _tc_big2_body,
        grid=(HBLK,),
        in_specs=[pl.BlockSpec(memory_space=pl.ANY), hi_spec, c_spec]
        + small_specs,
        out_specs=hi_spec,
        out_shape=jax.ShapeDtypeStruct((EXER_N, KN), f32),
        input_output_aliases={0: 0},
        name="exercise_layer_tc_b",
    )(o1, x, C1, zk_ek, s_ek, v2, web, be1)


def kernel(exer_emb, kn_emb, dir_edges, undir_edges, ek_edges,
           W_d, a_d, W_u, a_u, W_ek, a_ek, Wk1, bk1, Wk2, bk2, We1, be1):
    i32 = jnp.int32
    f32 = jnp.float32

    npad = EPAD - ek_edges.shape[1]
    pad_col = jnp.concatenate([
        jnp.full((1, npad), EXER_N, i32),
        jnp.full((1, npad), CPAD, i32),
    ], axis=0)
    ek_pad = jnp.concatenate([ek_edges.astype(i32), pad_col], axis=1)

    cek0_flat, C_d_flat, C_u_flat = _sc_histograms_half0(
        ek_pad[0], ek_pad[1],
        dir_edges[0].astype(i32), dir_edges[1].astype(i32),
        undir_edges[0].astype(i32), undir_edges[1].astype(i32))
    cek1_flat = _sc_histograms_half1(ek_pad[0], ek_pad[1])
    C0 = cek0_flat.reshape(2 * Q, KN)
    C1 = cek1_flat.reshape(2 * Q, KN)
    C_d = C_d_flat.reshape(KN, KN)
    C_u = C_u_flat.reshape(KN, KN)

    col = lambda v: v.reshape(KN, 1).astype(f32)
    row = lambda v: v.reshape(1, KN).astype(f32)
    small_args = (
        W_d.T, row(a_d[:KN, 0]), col(a_d[KN:, 0]),
        W_u.T, row(a_u[:KN, 0]), col(a_u[KN:, 0]),
        W_ek.T, row(a_ek[:KN, 0]), col(a_ek[KN:, 0]),
        col(Wk1[0, :KN]), col(Wk1[0, KN:]), bk1.reshape(1, 1),
        col(Wk2[0, :KN]), col(Wk2[0, KN:]), bk2.reshape(1, 1),
    )
    kn_out, zk_ek, s_ek, v_ek = _tc_small(kn_emb, C_d, C_u, small_args)

    v2 = jnp.concatenate([v_ek, col(We1[0, :KN])], axis=1)
    exer_out = _tc_big(exer_emb, C0, C1, zk_ek, s_ek, v2,
                       col(We1[0, KN:]), be1.reshape(1, 1))
    return exer_out, kn_out


# speedup vs baseline: 1.0585x; 1.0585x over previous
import jax
import jax.numpy as jnp
from jax import lax
from jax.experimental import pallas as pl
from jax.experimental.pallas import tpu as pltpu
from jax.experimental.pallas import tpu_sc as plsc

KN = 128
EXER_N = 50000

NSUB = 16
NCORE = 2
Q = 12544
CPAD = 4 * Q
ROWS_PER_TILE = Q // NSUB
CHUNK = 2048
NCHUNK = 13
EPAD = NSUB * NCHUNK * CHUNK
GARB = Q * 128
HISTW = (Q + 16) * 128
ZCH = 2512
E_DIR = 2048
E_UND = 4096


def _sc_hist_common(half, ek_src, ek_dst, dir_src, dir_dst, und_src, und_dst,
                    cek_out, cd_out, cu_out,
                    hist, hist_d, hist_u,
                    srcb0, srcb1, dstb0, dstb1, idxb0, idxb1, onesb, zb,
                    lsem0, lsem1, ssem0, ssem1):
    small = half == 0
    c = lax.axis_index("c")
    s = lax.axis_index("s")
    srcb = (srcb0, srcb1)
    dstb = (dstb0, dstb1)
    idxb = (idxb0, idxb1)
    lsem = (lsem0, lsem1)
    ssem = (ssem0, ssem1)

    if small:
        zero16 = jnp.zeros((16,), jnp.float32)
        one16 = jnp.ones((16,), jnp.float32)

        def init_body(i, _):
            zb[pl.ds(i * 16, 16)] = zero16
            return 0
        lax.fori_loop(0, ZCH // 16, init_body, 0)

        def ones_body(i, _):
            onesb[pl.ds(i * 16, 16)] = one16
            return 0
        lax.fori_loop(0, CHUNK // 16, ones_body, 0)

    lane = lax.iota(jnp.int32, 16)

    def compute_idx(sb, db, ib, base):
        def row_body(r, _):
            for kk in range(8):
                o = r * 128 + kk * 16
                d16 = db[pl.ds(o, 16)]
                s16 = sb[pl.ds(o, 16)]
                if base is None:
                    idx = d16 * 128 + s16
                else:
                    loc = d16 * 128 + (s16 - EXER_N) - base
                    ok = jnp.logical_and(loc >= 0, loc < Q * 128)
                    idx = jnp.where(ok, loc, GARB + o + lane)
                ib[pl.ds(o, 16)] = idx
            return 0
        lax.fori_loop(0, 16, row_body, 0)

    def scatter_pass(src_ref, dst_ref, qbase, tile_off):
        loads = [None] * NCHUNK
        streams = [None] * NCHUNK

        def start_load(t):
            off = tile_off + t * CHUNK
            b = t % 2
            loads[t] = (
                pltpu.async_copy(src_ref.at[pl.ds(off, CHUNK)], srcb[b], lsem[b]),
                pltpu.async_copy(dst_ref.at[pl.ds(off, CHUNK)], dstb[b], lsem[b]),
            )

        start_load(0)
        for t in range(NCHUNK):
            b = t % 2
            if t + 1 < NCHUNK:
                start_load(t + 1)
            for d in loads[t]:
                d.wait()
            if t >= 2:
                streams[t - 2].wait()
            compute_idx(srcb[b], dstb[b], idxb[b], qbase)
            streams[t] = pltpu.async_copy(onesb, hist.at[idxb[b]], ssem[b],
                                          add=True)
        streams[NCHUNK - 2].wait()
        streams[NCHUNK - 1].wait()

    def scatter_small(src_ref, dst_ref, off, target):
        pltpu.sync_copy(src_ref.at[pl.ds(off, CHUNK)], srcb0)
        pltpu.sync_copy(dst_ref.at[pl.ds(off, CHUNK)], dstb0)
        compute_idx(srcb0, dstb0, idxb0, None)
        pltpu.sync_copy(onesb, target.at[idxb0], add=True)

    quarter = 2 * half + c
    qbase = quarter * (Q * 128)

    zcopies = [pltpu.async_copy(
        zb, hist.at[pl.ds(s * (HISTW // 16) + j * ZCH, ZCH)],
        lsem[j % 2]) for j in range(40)]
    for d in zcopies:
        d.wait()
    if small:
        @pl.when(jnp.logical_and(c == 0, s == 2))
        def _():
            for j in range(8):
                pltpu.sync_copy(zb.at[pl.ds(0, 2048)],
                                hist_d.at[pl.ds(j * 2048, 2048)])

        @pl.when(jnp.logical_and(c == 0, s == 3))
        def _():
            for j in range(8):
                pltpu.sync_copy(zb.at[pl.ds(0, 2048)],
                                hist_u.at[pl.ds(j * 2048, 2048)])
    plsc.subcore_barrier()

    scatter_pass(ek_src, ek_dst, qbase, s * NCHUNK * CHUNK)

    if small:
        @pl.when(jnp.logical_and(c == 0, s == 0))
        def _():
            scatter_small(dir_src, dir_dst, 0, hist_d)

        @pl.when(jnp.logical_and(c == 0, s == 1))
        def _():
            scatter_small(und_src, und_dst, 0, hist_u)
            scatter_small(und_src, und_dst, CHUNK, hist_u)
    plsc.subcore_barrier()

    nw = ROWS_PER_TILE * 128
    pltpu.sync_copy(hist.at[pl.ds(s * nw, nw)],
                    cek_out.at[pl.ds(quarter * Q * 128 + s * nw, nw)])
    if small:
        @pl.when(jnp.logical_and(c == 0, s == 2))
        def _():
            pltpu.sync_copy(hist_d, cd_out)

        @pl.when(jnp.logical_and(c == 0, s == 3))
        def _():
            pltpu.sync_copy(hist_u, cu_out)


def _sc_body(ek_src, ek_dst, dir_src, dir_dst, und_src, und_dst,
             cek_out, cd_out, cu_out,
             hist, hist_d, hist_u,
             srcb0, srcb1, dstb0, dstb1, idxb0, idxb1, onesb, zb,
             lsem0, lsem1, ssem0, ssem1):
    for half in range(2):
        if half:
            plsc.subcore_barrier()
        _sc_hist_common(half, ek_src, ek_dst, dir_src, dir_dst, und_src,
                        und_dst, cek_out, cd_out, cu_out,
                        hist, hist_d, hist_u,
                        srcb0, srcb1, dstb0, dstb1, idxb0, idxb1, onesb, zb,
                        lsem0, lsem1, ssem0, ssem1)


_SC_TILE_SCRATCH = [
    pltpu.VMEM((CHUNK,), jnp.int32),
    pltpu.VMEM((CHUNK,), jnp.int32),
    pltpu.VMEM((CHUNK,), jnp.int32),
    pltpu.VMEM((CHUNK,), jnp.int32),
    pltpu.VMEM((CHUNK,), jnp.int32),
    pltpu.VMEM((CHUNK,), jnp.int32),
    pltpu.VMEM((CHUNK,), jnp.float32),
    pltpu.VMEM((ZCH,), jnp.float32),
    pltpu.SemaphoreType.DMA,
    pltpu.SemaphoreType.DMA,
    pltpu.SemaphoreType.DMA,
    pltpu.SemaphoreType.DMA,
]


def _sc_mesh():
    return plsc.VectorSubcoreMesh(core_axis_name="c", subcore_axis_name="s",
                                  num_cores=NCORE, num_subcores=NSUB)


def _sc_histograms(ek_src, ek_dst, dir_src, dir_dst, und_src, und_dst):
    f32 = jnp.float32
    return pl.kernel(
        _sc_body,
        out_type=(
            jax.ShapeDtypeStruct((CPAD * 128,), f32),
            jax.ShapeDtypeStruct((KN * KN,), f32),
            jax.ShapeDtypeStruct((KN * KN,), f32),
        ),
        mesh=_sc_mesh(),
        scratch_types=[
            pltpu.VMEM_SHARED((HISTW,), f32),
            pltpu.VMEM_SHARED((KN * KN,), f32),
            pltpu.VMEM_SHARED((KN * KN,), f32),
        ] + _SC_TILE_SCRATCH,
        name="edge_histograms_sc",
    )(ek_src, ek_dst, dir_src, dir_dst, und_src, und_dst)


def _gat_dense(z, s_row, t_col, cnt):
    u = t_col + s_row
    f = jnp.maximum(u, jnp.float32(0.01) * u)
    a = t_col + jnp.max(s_row, axis=1, keepdims=True)
    m = jnp.maximum(a, jnp.float32(0.01) * a)
    p = cnt * jnp.exp(f - m)
    den = jnp.sum(p, axis=1, keepdims=True)
    q = jnp.dot(p, z, preferred_element_type=jnp.float32)
    return q * (jnp.float32(1.0) / (den + jnp.float32(1e-30)))


def _nt(a, b):
    return lax.dot_general(a, b, (((1,), (1,)), ((), ())),
                           preferred_element_type=jnp.float32)


def _tc_small_body(kn_ref, cd_ref, cu_ref, wdt_ref, a1d_ref, a2d_ref,
                   wut_ref, a1u_ref, a2u_ref, wekt_ref, a1ek_ref, a2ek_ref,
                   wk1a_ref, wk1b_ref, bk1_ref, wk2a_ref, wk2b_ref, bk2_ref,
                   kn_out_ref, zkek_ref, sek_ref, vek_ref):
    A = kn_ref[...]

    zd = jnp.dot(A, wdt_ref[...], preferred_element_type=jnp.float32)
    B = _gat_dense(zd, _nt(a1d_ref[...], zd),
                   jnp.dot(zd, a2d_ref[...], preferred_element_type=jnp.float32),
                   cd_ref[...])

    zu = jnp.dot(A, wut_ref[...], preferred_element_type=jnp.float32)
    C = _gat_dense(zu, _nt(a1u_ref[...], zu),
                   jnp.dot(zu, a2u_ref[...], preferred_element_type=jnp.float32),
                   cu_ref[...])

    s1 = (jnp.dot(A, wk1a_ref[...], preferred_element_type=jnp.float32)
          + jnp.dot(B, wk1b_ref[...], preferred_element_type=jnp.float32)
          + bk1_ref[...])
    s2 = (jnp.dot(A, wk2a_ref[...], preferred_element_type=jnp.float32)
          + jnp.dot(C, wk2b_ref[...], preferred_element_type=jnp.float32)
          + bk2_ref[...])
    mm = jnp.maximum(s1, s2)
    e1 = jnp.exp(s1 - mm)
    e2 = jnp.exp(s2 - mm)
    kn_out_ref[...] = A + (e1 * B + e2 * C) / (e1 + e2)

    zek = jnp.dot(A, wekt_ref[...], preferred_element_type=jnp.float32)
    zkek_ref[...] = zek
    sek_ref[...] = _nt(a1ek_ref[...], zek)
    vek_ref[...] = jnp.dot(wekt_ref[...], a2ek_ref[...],
                           preferred_element_type=jnp.float32)


def _tc_small(kn_emb, C_d, C_u, args):
    f32 = jnp.float32
    out_shape = (
        jax.ShapeDtypeStruct((KN, KN), f32),
        jax.ShapeDtypeStruct((KN, KN), f32),
        jax.ShapeDtypeStruct((1, KN), f32),
        jax.ShapeDtypeStruct((KN, 1), f32),
    )
    return pl.pallas_call(
        _tc_small_body,
        out_shape=out_shape,
        name="knowledge_layers_tc",
    )(kn_emb, C_d, C_u, *args)


BLK = 1024
NBLK = CPAD // BLK


def _tc_big_body(x_ref, c_ref, zk_ref, s_ref, v2_ref, wb_ref, be_ref,
                 o_ref):
    X = x_ref[...]
    XV = jnp.dot(X, v2_ref[...], preferred_element_type=jnp.float32)
    t = XV[:, 0:1]
    sa = XV[:, 1:2]
    s = s_ref[...]
    u = t + s
    f = jnp.maximum(u, jnp.float32(0.01) * u)
    a = t + jnp.max(s, axis=1, keepdims=True)
    m = jnp.maximum(a, jnp.float32(0.01) * a)
    p = c_ref[...] * jnp.exp(f - m)
    den = jnp.sum(p, axis=1, keepdims=True)
    q = jnp.dot(p, zk_ref[...], preferred_element_type=jnp.float32)
    Be = q * (jnp.float32(1.0) / (den + jnp.float32(1e-30)))
    se = (sa + jnp.dot(Be, wb_ref[...], preferred_element_type=jnp.float32)
          + be_ref[...])
    o_ref[...] = X + se * Be


def _tc_big(x, C_ek, zk_ek, s_ek, v2, web, be1):
    f32 = jnp.float32
    row_spec = pl.BlockSpec((BLK, KN), lambda i: (i, 0))
    full = lambda shp: pl.BlockSpec(shp, lambda i: (0, 0))
    return pl.pallas_call(
        _tc_big_body,
        grid=(NBLK,),
        in_specs=[
            row_spec,
            row_spec,
            full((KN, KN)),
            full((1, KN)),
            full((KN, 2)),
            full((KN, 1)),
            full((1, 1)),
        ],
        out_specs=row_spec,
        out_shape=jax.ShapeDtypeStruct((EXER_N, KN), f32),
        name="exercise_layer_tc",
    )(x, C_ek, zk_ek, s_ek, v2, web, be1)


def kernel(exer_emb, kn_emb, dir_edges, undir_edges, ek_edges,
           W_d, a_d, W_u, a_u, W_ek, a_ek, Wk1, bk1, Wk2, bk2, We1, be1):
    i32 = jnp.int32
    f32 = jnp.float32

    npad = EPAD - ek_edges.shape[1]
    pad_col = jnp.concatenate([
        jnp.full((1, npad), EXER_N, i32),
        jnp.full((1, npad), CPAD, i32),
    ], axis=0)
    ek_pad = jnp.concatenate([ek_edges.astype(i32), pad_col], axis=1)

    C_ek_flat, C_d_flat, C_u_flat = _sc_histograms(
        ek_pad[0], ek_pad[1],
        dir_edges[0].astype(i32), dir_edges[1].astype(i32),
        undir_edges[0].astype(i32), undir_edges[1].astype(i32))
    C_ek = C_ek_flat.reshape(CPAD, KN)
    C_d = C_d_flat.reshape(KN, KN)
    C_u = C_u_flat.reshape(KN, KN)

    col = lambda v: v.reshape(KN, 1).astype(f32)
    row = lambda v: v.reshape(1, KN).astype(f32)
    small_args = (
        W_d.T, row(a_d[:KN, 0]), col(a_d[KN:, 0]),
        W_u.T, row(a_u[:KN, 0]), col(a_u[KN:, 0]),
        W_ek.T, row(a_ek[:KN, 0]), col(a_ek[KN:, 0]),
        col(Wk1[0, :KN]), col(Wk1[0, KN:]), bk1.reshape(1, 1),
        col(Wk2[0, :KN]), col(Wk2[0, KN:]), bk2.reshape(1, 1),
    )
    kn_out, zk_ek, s_ek, v_ek = _tc_small(kn_emb, C_d, C_u, small_args)

    v2 = jnp.concatenate([v_ek, col(We1[0, :KN])], axis=1)
    exer_out = _tc_big(exer_emb, C_ek, zk_ek, s_ek, v2,
                       col(We1[0, KN:]), be1.reshape(1, 1))
    return exer_out, kn_out
